# baseline (speedup 1.0000x reference)
import dataclasses
import functools

import numpy as np

import jax
import jax.numpy as jnp
from jax.experimental import pallas as pl
from jax.experimental.pallas import tpu as pltpu
from jax.experimental.pallas import tpu_sc as plsc

_RBF_OFF = np.linspace(0.0, 10.0, 32).astype(np.float32)
_RBF_STEP = np.float32(_RBF_OFF[1] - _RBF_OFF[0])
_RBF_COEFF = np.float32(-0.5 / (_RBF_OFF[1] - _RBF_OFF[0]) ** 2)


def _snorm(x, axis):
    x = jnp.maximum(jnp.abs(x), 1e-10)
    return jnp.sqrt(jnp.sum(x * x, axis=axis))


def _ln(x, w, b, eps=1e-5):
    mu = jnp.mean(x, axis=-1, keepdims=True)
    var = jnp.mean((x - mu) ** 2, axis=-1, keepdims=True)
    return (x - mu) / jnp.sqrt(var + eps) * w + b


def _silu(x):
    return x * jax.nn.sigmoid(x)


def _cross3(a, b):
    a0, a1, a2 = a[..., 0, :], a[..., 1, :], a[..., 2, :]
    b0, b1, b2 = b[..., 0, :], b[..., 1, :], b[..., 2, :]
    return jnp.stack([a1 * b2 - a2 * b1, a2 * b0 - a0 * b2, a0 * b1 - a1 * b0],
                     axis=-2)


def _mm(x, w):
    xf = x.reshape(-1, x.shape[-1])
    o = jax.lax.dot_general(xf, w, (((1,), (0,)), ((), ())),
                            preferred_element_type=jnp.float32)
    return o.reshape(x.shape[:-1] + (w.shape[-1],))


def _gvp(Hs, Vs, Wv, W1, b1, W2, b2, lnw, lnb, d_hidden, d_s_out):
    Vt = jnp.swapaxes(Vs, -1, -2)
    Vr = jnp.concatenate([Vt[..., 1:], Vt[..., :1]], axis=-1)
    Vc = jnp.concatenate([Vt, _cross3(Vt, Vr)], axis=-1)
    Vp = _mm(Vc, Wv)
    V1, V2 = Vp[..., :d_hidden], Vp[..., d_hidden:]
    scaler = jnp.concatenate([Hs, _snorm(V1, axis=-2)], axis=-1)
    h = _silu(_mm(scaler, W1) + b1)
    so = _mm(h, W2) + b2
    Ho, V_up = so[..., :d_s_out], so[..., d_s_out:]
    Vo = _ln(V_up, lnw, lnb)[..., None, :] * V2
    return Ho, jnp.swapaxes(Vo, -1, -2)


def _node_body(h_ref, v_ref, x_ref, wv_ref, w1_ref, b1_ref, w2_ref, b2_ref,
               lnw_ref, lnb_ref, r_ref, c1_ref, c2_ref):
    TN = h_ref.shape[0]
    Hb, Vb, Xb = h_ref[...], v_ref[...], x_ref[...]
    Hh = Hb.reshape(TN, 4, 4, 32).transpose(0, 2, 1, 3)
    Vh = Vb.reshape(TN, 4, 4, 4, 3).transpose(0, 2, 1, 3, 4)
    Hqk, Vqk = _gvp(Hh, Vh, wv_ref[...], w1_ref[...], b1_ref[...][0],
                    w2_ref[...], b2_ref[...][0], lnw_ref[...][0],
                    lnb_ref[...][0], 64, 64)
    q = jnp.concatenate([Hqk[..., :32], Vqk[..., :4, :].reshape(TN, 4, 4, 12)],
                        axis=-1)
    k = jnp.concatenate([Hqk[..., 32:], Vqk[..., 4:, :].reshape(TN, 4, 4, 12)],
                        axis=-1)
    pad52 = jnp.zeros((TN, 52), jnp.float32)
    pad64 = jnp.zeros((TN, 64), jnp.float32)
    r_ref[...] = jnp.concatenate(
        [q.reshape(TN, 704), Xb.reshape(TN, 12), pad52], axis=-1)
    c1_ref[...] = jnp.concatenate([k.reshape(TN, 704), pad64], axis=-1)
    c2_ref[...] = jnp.concatenate(
        [Hh.reshape(TN, 512), Vh.reshape(TN, 192), Xb.reshape(TN, 12), pad52],
        axis=-1)


def _node_pack(H, V, X, qk_Wv, qk_W1, qk_b1, qk_W2, qk_b2, qk_ln_w, qk_ln_b):
    N = H.shape[0]
    TN = 80 if N % 80 == 0 else N
    g = N // TN
    ws = (qk_Wv, qk_W1, qk_b1, qk_W2, qk_b2, qk_ln_w, qk_ln_b)
    wspec = [pl.BlockSpec(w.shape, lambda i: (0,) * w.ndim) for w in ws]
    return pl.pallas_call(
        _node_body,
        grid=(g,),
        in_specs=[
            pl.BlockSpec((TN, 4, 128), lambda i: (i, 0, 0)),
            pl.BlockSpec((TN, 4, 16, 3), lambda i: (i, 0, 0, 0)),
            pl.BlockSpec((TN, 4, 3), lambda i: (i, 0, 0)),
        ] + wspec,
        out_specs=[
            pl.BlockSpec((TN, 768), lambda i: (i, 0)),
            pl.BlockSpec((TN, 768), lambda i: (i, 0)),
            pl.BlockSpec((TN, 768), lambda i: (i, 0)),
        ],
        out_shape=[
            jax.ShapeDtypeStruct((N, 768), jnp.float32),
            jax.ShapeDtypeStruct((N, 768), jnp.float32),
            jax.ShapeDtypeStruct((N, 768), jnp.float32),
        ],
    )(H, V, X, *ws)


def _sc_gather(table, idx):
    E = idx.shape[0]
    W = table.shape[1]
    NW = 32
    CH = 40
    e_per_w = E // NW
    n_chunks = e_per_w // CH
    mesh = plsc.VectorSubcoreMesh(core_axis_name="c", subcore_axis_name="s")

    @functools.partial(
        pl.kernel,
        out_type=jax.ShapeDtypeStruct((E, W), jnp.float32),
        mesh=mesh,
        scratch_types=[
            pltpu.VMEM((e_per_w,), jnp.int32),
            pltpu.VMEM((CH, W), jnp.float32),
        ],
    )
    def k(tab_hbm, i_hbm, o_hbm, idx_v, rows_v):
        cid = jax.lax.axis_index("c")
        sid = jax.lax.axis_index("s")
        wid = sid * 2 + cid
        base = wid * e_per_w
        pltpu.sync_copy(i_hbm.at[pl.ds(base, e_per_w)], idx_v)

        @pl.loop(0, n_chunks)
        def _(j):
            pltpu.sync_copy(tab_hbm.at[idx_v.at[pl.ds(j * CH, CH)]], rows_v)
            pltpu.sync_copy(rows_v, o_hbm.at[pl.ds(base + j * CH, CH)])

    return k(table, idx)


def _edge_body(gr_ref, gk_ref, gc_ref, wv_ref, w1_ref, b1_ref, w2_ref,
               b2_ref, lnw_ref, lnb_ref, wr_ref, o_ref):
    TE = gr_ref.shape[0]
    gr, gk, gc = gr_ref[...], gk_ref[...], gc_ref[...]
    q = gr[:, :704].reshape(TE, 4, 4, 44)
    Xr = gr[:, 704:716].reshape(TE, 4, 3)
    k = gk[:, :704].reshape(TE, 4, 4, 44)
    Hc = gc[:, :512].reshape(TE, 4, 4, 32)
    Vc = gc[:, 512:704].reshape(TE, 4, 4, 4, 3)
    Xc = gc[:, 704:716].reshape(TE, 4, 3)

    Xij = Xr[:, :, None, :] - Xc[:, None, :, :]
    D = _snorm(Xij, axis=-1)
    off = (jax.lax.broadcasted_iota(jnp.int32, (1, 1, 1, 32), 3)
           .astype(jnp.float32) * _RBF_STEP)
    R = jnp.exp(_RBF_COEFF * (D[..., None] - off) ** 2)
    w32 = wr_ref[...].reshape(1, 1, 1, 32)
    rw = (R * w32).reshape(TE, 4, 4, 4, 8).sum(-1)
    rw_t = rw.transpose(0, 3, 1, 2)
    qk = (q[:, :, :, None, :] * k[:, :, None, :, :]).sum(-1)
    alpha = _silu(qk * rw_t)

    Rh = R.reshape(TE, 4, 4, 4, 8)
    a_ijh = alpha.transpose(0, 2, 3, 1)
    Hagg = (a_ijh[..., None] * Rh).sum(1).transpose(0, 2, 1, 3)
    Xagg = (a_ijh[:, :, :, :, None] * Xij[:, :, :, None, :]).sum(1)
    Xagg = Xagg.transpose(0, 2, 1, 3)

    H_in = jnp.concatenate([Hc, Hagg], axis=-1)
    V_in = jnp.concatenate([Vc, Xagg[..., None, :]], axis=-2)
    Hv, Vv = _gvp(H_in, V_in, wv_ref[...], w1_ref[...], b1_ref[...][0],
                  w2_ref[...], b2_ref[...][0], lnw_ref[...][0],
                  lnb_ref[...][0], 64, 32)
    Vv12 = Vv.reshape(TE, 4, 4, 12)
    Hw = (alpha[..., None] * Hv[:, :, None, :, :]).sum(3)
    Vw = (alpha[..., None] * Vv12[:, :, None, :, :]).sum(3)
    packed = jnp.concatenate(
        [Hw.transpose(0, 2, 1, 3).reshape(TE, 512),
         Vw.transpose(0, 2, 1, 3).reshape(TE, 192),
         jnp.zeros((TE, 64), jnp.float32)], axis=-1)
    o_ref[...] = packed.T


def _edge_compute(G_r, G_k, G_c, v_Wv, v_W1, v_b1, v_W2, v_b2, v_ln_w,
                  v_ln_b, w32):
    E = G_r.shape[0]
    TE = 128 if E % 128 == 0 else E
    g = E // TE
    ws = (v_Wv, v_W1, v_b1, v_W2, v_b2, v_ln_w, v_ln_b, w32)
    wspec = [pl.BlockSpec(w.shape, lambda i: (0,) * w.ndim) for w in ws]
    return pl.pallas_call(
        _edge_body,
        grid=(g,),
        in_specs=[
            pl.BlockSpec((TE, 768), lambda i: (i, 0)),
            pl.BlockSpec((TE, 768), lambda i: (i, 0)),
            pl.BlockSpec((TE, 768), lambda i: (i, 0)),
        ] + wspec,
        out_specs=pl.BlockSpec((768, TE), lambda i: (0, i)),
        out_shape=jax.ShapeDtypeStruct((768, E), jnp.float32),
    )(G_r, G_k, G_c, *ws)


def _sc_scatter_add(edge_out_t, row, init_t):
    E = edge_out_t.shape[1]
    N = init_t.shape[1]
    F, FS, CH = 768, 24, 640
    n_chunks = E // CH
    mesh = plsc.VectorSubcoreMesh(core_axis_name="c", subcore_axis_name="s")
    cp = pltpu.CompilerParams()
    if "needs_layout_passes" in pltpu.CompilerParams.__dataclass_fields__:
        cp = dataclasses.replace(cp, needs_layout_passes=False)

    @functools.partial(
        pl.kernel,
        out_type=jax.ShapeDtypeStruct((F, N), jnp.float32),
        mesh=mesh,
        compiler_params=cp,
        scratch_types=[
            pltpu.VMEM((FS, N), jnp.float32),
            pltpu.VMEM((FS, CH), jnp.float32),
            pltpu.VMEM((CH,), jnp.int32),
        ],
    )
    def k(eo_hbm, idx_hbm, init_hbm, out_hbm, acc, buf, ibuf):
        cid = jax.lax.axis_index("c")
        sid = jax.lax.axis_index("s")
        wid = sid * 2 + cid
        f0 = wid * FS
        pltpu.sync_copy(init_hbm.at[pl.ds(f0, FS)], acc)

        @pl.loop(0, n_chunks)
        def _(j):
            base = j * CH
            pltpu.sync_copy(eo_hbm.at[pl.ds(f0, FS), pl.ds(base, CH)], buf)
            pltpu.sync_copy(idx_hbm.at[pl.ds(base, CH)], ibuf)

            @pl.loop(0, CH // 16)
            def _(g):
                idx_g = ibuf[pl.ds(g * 16, 16)]
                for f in range(FS):
                    vals = buf[f, pl.ds(g * 16, 16)]
                    f_vec = jnp.full((16,), f, jnp.int32)
                    plsc.addupdate_scatter(acc, [f_vec, idx_g], vals)

        pltpu.sync_copy(acc, out_hbm.at[pl.ds(f0, FS)])

    return k(edge_out_t, row, init_t)


def kernel(H, V, X, mask, edge_index, qk_Wv, qk_W1, qk_b1, qk_W2, qk_b2,
           qk_ln_w, qk_ln_b, v_Wv, v_W1, v_b1, v_W2, v_b2, v_ln_w, v_ln_b,
           WR):
    N = H.shape[0]
    row, col = edge_index[0], edge_index[1]

    R_pack, C1, C2 = _node_pack(
        H, V, X, qk_Wv, qk_W1, qk_b1.reshape(1, 64), qk_W2,
        qk_b2.reshape(1, 72), qk_ln_w.reshape(1, 8), qk_ln_b.reshape(1, 8))

    G_r = _sc_gather(R_pack, row)
    G_k = _sc_gather(C1, col)
    G_c = _sc_gather(C2, col)

    w32 = jnp.tile(WR[:, 0], 4).reshape(1, 32)
    edge_out = _edge_compute(
        G_r, G_k, G_c, v_Wv, v_W1, v_b1.reshape(1, 64), v_W2,
        v_b2.reshape(1, 36), v_ln_w.reshape(1, 4), v_ln_b.reshape(1, 4), w32)

    init_t = jnp.concatenate(
        [H.reshape(N, 512), V.reshape(N, 192),
         jnp.zeros((N, 64), jnp.float32)], axis=1).T
    acc = _sc_scatter_add(edge_out, row, init_t).T

    H_out = acc[:, :512].reshape(N, 4, 128)
    V_out = acc[:, 512:704].reshape(N, 4, 16, 3)
    return H_out, V_out, X
